# Initial kernel scaffold; baseline (speedup 1.0000x reference)
import jax
import jax.numpy as jnp
from jax.experimental import pallas as pl


def kernel(x, templates, committed, category_labels, category_counts, num_committed):
    raise NotImplementedError("write your pallas kernel here")


# speedup vs baseline: 1.4731x; 1.4731x over previous
import functools

import jax
import jax.numpy as jnp
from jax.experimental import pallas as pl

INPUT_DIM = 128
TWO_D = 2 * INPUT_DIM
K = 512
B = 512
CHOICE_ALPHA = 0.001
GAMMA = 0.01
NUM_CLASSES = 10
C_PAD = 16

BT = 128
KT = 128


def _choice_body(x_ref, tT_ref, counts_ref, comm_ref, out_ref):
    bb = pl.program_id(0)

    xf = x_ref[...]
    mn = jnp.min(xf)
    mx = jnp.max(xf)
    scale = mx - mn + 1e-10
    xb = x_ref[pl.ds(bb * BT, BT), :]
    xn = (xb - mn) / scale
    xc2 = 1.0 - xn

    tT = tT_ref[...]

    acc = jnp.zeros((BT, KT), dtype=jnp.float32)
    for d in range(INPUT_DIM):
        a1 = jax.lax.broadcast_in_dim(xn[:, d], (BT, KT), (0,))
        a2 = jax.lax.broadcast_in_dim(xc2[:, d], (BT, KT), (0,))
        b1 = jax.lax.broadcast_in_dim(tT[d, :], (BT, KT), (1,))
        b2 = jax.lax.broadcast_in_dim(tT[d + INPUT_DIM, :], (BT, KT), (1,))
        acc = acc + jnp.minimum(a1, b1) + jnp.minimum(a2, b2)

    s_t = jnp.sum(tT, axis=0, keepdims=True)
    denom = CHOICE_ALPHA + s_t + GAMMA * counts_ref[0:1, :]
    cv = acc / denom
    cv = jnp.where(comm_ref[0:1, :] > 0.0, cv, -jnp.inf)
    out_ref[...] = cv


def _post_body(choice_ref, labels_ref, comm_ref, out_ref):
    cv = choice_ref[...]
    row_max = jnp.max(cv, axis=1, keepdims=True)
    iota_k = jax.lax.broadcasted_iota(jnp.int32, (B, K), 1)
    masked_idx = jnp.where(cv == row_max, iota_k, K)
    best = jnp.min(masked_idx, axis=1, keepdims=True)
    best_oh = (iota_k == best).astype(jnp.float32)

    cls_iota = jax.lax.broadcasted_iota(jnp.int32, (K, C_PAD), 1)
    lab_oh = (labels_ref[...] == cls_iota).astype(jnp.float32)

    cvz = jnp.where(comm_ref[0:1, :] > 0.0, cv, 0.0)
    cls_sums = jnp.dot(cvz, lab_oh, preferred_element_type=jnp.float32)
    pred_oh = jnp.dot(best_oh, lab_oh, preferred_element_type=jnp.float32)
    out_ref[...] = pred_oh * cls_sums


@jax.jit
def _run(x, templates, comm_f, labels2d, counts_f):
    tT = templates.T
    counts_b = jnp.broadcast_to(counts_f[None, :], (8, K))
    comm_b = jnp.broadcast_to(comm_f[None, :], (8, K))

    choice = pl.pallas_call(
        _choice_body,
        grid=(B // BT, K // KT),
        in_specs=[
            pl.BlockSpec((B, INPUT_DIM), lambda bb, kb: (0, 0)),
            pl.BlockSpec((TWO_D, KT), lambda bb, kb: (0, kb)),
            pl.BlockSpec((8, KT), lambda bb, kb: (0, kb)),
            pl.BlockSpec((8, KT), lambda bb, kb: (0, kb)),
        ],
        out_specs=pl.BlockSpec((BT, KT), lambda bb, kb: (bb, kb)),
        out_shape=jax.ShapeDtypeStruct((B, K), jnp.float32),
    )(x, tT, counts_b, comm_b)

    logits_p = pl.pallas_call(
        _post_body,
        in_specs=[
            pl.BlockSpec((B, K), lambda: (0, 0)),
            pl.BlockSpec((K, 1), lambda: (0, 0)),
            pl.BlockSpec((8, K), lambda: (0, 0)),
        ],
        out_specs=pl.BlockSpec((B, C_PAD), lambda: (0, 0)),
        out_shape=jax.ShapeDtypeStruct((B, C_PAD), jnp.float32),
    )(choice, labels2d, comm_b)
    return logits_p[:, :NUM_CLASSES]


def kernel(x, templates, committed, category_labels, category_counts, num_committed):
    comm_f = committed.astype(jnp.float32)
    counts_f = category_counts.astype(jnp.float32)
    labels2d = category_labels.reshape(K, 1)
    return _run(x, templates, comm_f, labels2d, counts_f)


# speedup vs baseline: 3.9090x; 2.6536x over previous
import functools

import jax
import jax.numpy as jnp
from jax.experimental import pallas as pl

INPUT_DIM = 128
TWO_D = 2 * INPUT_DIM
K = 512
B = 512
CHOICE_ALPHA = 0.001
GAMMA = 0.01
NUM_CLASSES = 10
C_PAD = 16

BT = 128
KC = 128
NC = K // KC
G = 2


def _choice_body(x_ref, tT_ref, counts_ref, comm_ref, out_ref):
    bb = pl.program_id(0)

    xf = x_ref[...]
    mn = jnp.min(xf)
    mx = jnp.max(xf)
    scale = mx - mn + 1e-10
    s_t = jnp.sum(tT_ref[...], axis=0, keepdims=True)
    denom = CHOICE_ALPHA + s_t + GAMMA * counts_ref[0:1, :]
    inv_denom = 1.0 / denom
    comm = comm_ref[0:1, :] > 0.0

    R = BT // 8
    for g in range(R // G):
        xng = (x_ref[pl.ds(bb * BT + 8 * G * g, 8 * G), :] - mn) / scale
        accg = [[jnp.zeros((8, KC), dtype=jnp.float32) for _ in range(NC)]
                for _ in range(G)]
        for d in range(INPUT_DIM):
            a1s = []
            a2s = []
            for j in range(G):
                a1 = jax.lax.broadcast_in_dim(
                    xng[8 * j : 8 * j + 8, d : d + 1], (8, KC), (0, 1)
                )
                a1s.append(a1)
                a2s.append(1.0 - a1)
            for c in range(NC):
                b1 = jnp.broadcast_to(tT_ref[d : d + 1, c * KC : (c + 1) * KC], (8, KC))
                b2 = jnp.broadcast_to(
                    tT_ref[d + INPUT_DIM : d + INPUT_DIM + 1, c * KC : (c + 1) * KC],
                    (8, KC),
                )
                for j in range(G):
                    accg[j][c] = (
                        accg[j][c] + jnp.minimum(a1s[j], b1) + jnp.minimum(a2s[j], b2)
                    )
        for j in range(G):
            for c in range(NC):
                cvj = jnp.where(
                    comm[:, c * KC : (c + 1) * KC],
                    accg[j][c] * inv_denom[:, c * KC : (c + 1) * KC],
                    -jnp.inf,
                )
                out_ref[pl.ds(8 * (G * g + j), 8), c * KC : (c + 1) * KC] = cvj


def _post_body(choice_ref, labels_ref, comm_ref, out_ref):
    cv = choice_ref[...]
    row_max = jnp.max(cv, axis=1, keepdims=True)
    iota_k = jax.lax.broadcasted_iota(jnp.int32, (B, K), 1)
    masked_idx = jnp.where(cv == row_max, iota_k, K)
    best = jnp.min(masked_idx, axis=1, keepdims=True)
    best_oh = (iota_k == best).astype(jnp.float32)

    cls_iota = jax.lax.broadcasted_iota(jnp.int32, (K, C_PAD), 1)
    lab_oh = (labels_ref[...] == cls_iota).astype(jnp.float32)

    cvz = jnp.where(comm_ref[0:1, :] > 0.0, cv, 0.0)
    cls_sums = jnp.dot(cvz, lab_oh, preferred_element_type=jnp.float32)
    pred_oh = jnp.dot(best_oh, lab_oh, preferred_element_type=jnp.float32)
    out_ref[...] = pred_oh * cls_sums


@jax.jit
def _run(x, templates, comm_f, labels2d, counts_f):
    tT = templates.T
    counts_b = jnp.broadcast_to(counts_f[None, :], (8, K))
    comm_b = jnp.broadcast_to(comm_f[None, :], (8, K))

    choice = pl.pallas_call(
        _choice_body,
        grid=(B // BT,),
        in_specs=[
            pl.BlockSpec((B, INPUT_DIM), lambda bb: (0, 0)),
            pl.BlockSpec((TWO_D, K), lambda bb: (0, 0)),
            pl.BlockSpec((8, K), lambda bb: (0, 0)),
            pl.BlockSpec((8, K), lambda bb: (0, 0)),
        ],
        out_specs=pl.BlockSpec((BT, K), lambda bb: (bb, 0)),
        out_shape=jax.ShapeDtypeStruct((B, K), jnp.float32),
    )(x, tT, counts_b, comm_b)

    logits_p = pl.pallas_call(
        _post_body,
        in_specs=[
            pl.BlockSpec((B, K), lambda: (0, 0)),
            pl.BlockSpec((K, 1), lambda: (0, 0)),
            pl.BlockSpec((8, K), lambda: (0, 0)),
        ],
        out_specs=pl.BlockSpec((B, C_PAD), lambda: (0, 0)),
        out_shape=jax.ShapeDtypeStruct((B, C_PAD), jnp.float32),
    )(choice, labels2d, comm_b)
    return logits_p[:, :NUM_CLASSES]


def kernel(x, templates, committed, category_labels, category_counts, num_committed):
    comm_f = committed.astype(jnp.float32)
    counts_f = category_counts.astype(jnp.float32)
    labels2d = category_labels.reshape(K, 1)
    return _run(x, templates, comm_f, labels2d, counts_f)


# speedup vs baseline: 4.0786x; 1.0434x over previous
import jax
import jax.numpy as jnp
from jax.experimental import pallas as pl
from jax.experimental.pallas import tpu as pltpu

INPUT_DIM = 128
TWO_D = 2 * INPUT_DIM
K = 512
B = 512
CHOICE_ALPHA = 0.001
GAMMA = 0.01
NUM_CLASSES = 10
C_PAD = 16

BT = 128
KC = 128
NC = K // KC
G = 2


def _body(x_ref, tT_ref, counts_ref, comm_ref, labels_ref,
          out_ref, xn_scr, choice_scr):
    bb = pl.program_id(0)

    @pl.when(bb == 0)
    def _():
        xf = x_ref[...]
        mn = jnp.min(xf)
        mx = jnp.max(xf)
        xn_scr[...] = (xf - mn) / (mx - mn + 1e-10)

    s_t = jnp.sum(tT_ref[...], axis=0, keepdims=True)
    denom = CHOICE_ALPHA + s_t + GAMMA * counts_ref[0:1, :]
    inv_denom = 1.0 / denom
    comm = comm_ref[0:1, :] > 0.0

    R = BT // 8
    for g in range(R // G):
        accg = [[jnp.zeros((8, KC), dtype=jnp.float32) for _ in range(NC)]
                for _ in range(G)]
        row0 = bb * BT + 8 * G * g
        for d in range(INPUT_DIM):
            a1s = []
            a2s = []
            for j in range(G):
                a1 = jax.lax.broadcast_in_dim(
                    xn_scr[pl.ds(row0 + 8 * j, 8), d : d + 1],
                    (8, KC), (0, 1),
                )
                a1s.append(a1)
                a2s.append(1.0 - a1)
            for c in range(NC):
                b1 = jnp.broadcast_to(tT_ref[d : d + 1, c * KC : (c + 1) * KC], (8, KC))
                b2 = jnp.broadcast_to(
                    tT_ref[d + INPUT_DIM : d + INPUT_DIM + 1, c * KC : (c + 1) * KC],
                    (8, KC),
                )
                for j in range(G):
                    accg[j][c] = (
                        accg[j][c] + jnp.minimum(a1s[j], b1) + jnp.minimum(a2s[j], b2)
                    )
        for j in range(G):
            for c in range(NC):
                cvj = jnp.where(
                    comm[:, c * KC : (c + 1) * KC],
                    accg[j][c] * inv_denom[:, c * KC : (c + 1) * KC],
                    -jnp.inf,
                )
                choice_scr[pl.ds(row0 + 8 * j, 8), c * KC : (c + 1) * KC] = cvj

    @pl.when(bb == pl.num_programs(0) - 1)
    def _():
        cv = choice_scr[...]
        row_max = jnp.max(cv, axis=1, keepdims=True)
        iota_k = jax.lax.broadcasted_iota(jnp.int32, (B, K), 1)
        masked_idx = jnp.where(cv == row_max, iota_k, K)
        best = jnp.min(masked_idx, axis=1, keepdims=True)
        best_oh = (iota_k == best).astype(jnp.float32)

        cls_iota = jax.lax.broadcasted_iota(jnp.int32, (K, C_PAD), 1)
        lab_oh = (labels_ref[...] == cls_iota).astype(jnp.float32)

        cvz = jnp.where(comm, cv, 0.0)
        cls_sums = jnp.dot(cvz, lab_oh, preferred_element_type=jnp.float32)
        pred_oh = jnp.dot(best_oh, lab_oh, preferred_element_type=jnp.float32)
        out_ref[...] = pred_oh * cls_sums


@jax.jit
def _run(x, templates, comm_f, labels2d, counts_f):
    tT = templates.T
    counts_b = jnp.broadcast_to(counts_f[None, :], (8, K))
    comm_b = jnp.broadcast_to(comm_f[None, :], (8, K))

    logits_p = pl.pallas_call(
        _body,
        grid=(B // BT,),
        in_specs=[
            pl.BlockSpec((B, INPUT_DIM), lambda bb: (0, 0)),
            pl.BlockSpec((TWO_D, K), lambda bb: (0, 0)),
            pl.BlockSpec((8, K), lambda bb: (0, 0)),
            pl.BlockSpec((8, K), lambda bb: (0, 0)),
            pl.BlockSpec((K, 1), lambda bb: (0, 0)),
        ],
        out_specs=pl.BlockSpec((B, C_PAD), lambda bb: (0, 0)),
        out_shape=jax.ShapeDtypeStruct((B, C_PAD), jnp.float32),
        scratch_shapes=[
            pltpu.VMEM((B, INPUT_DIM), jnp.float32),
            pltpu.VMEM((B, K), jnp.float32),
        ],
    )(x, tT, counts_b, comm_b, labels2d)
    return logits_p[:, :NUM_CLASSES]


def kernel(x, templates, committed, category_labels, category_counts, num_committed):
    comm_f = committed.astype(jnp.float32)
    counts_f = category_counts.astype(jnp.float32)
    labels2d = category_labels.reshape(K, 1)
    return _run(x, templates, comm_f, labels2d, counts_f)


# speedup vs baseline: 4.2198x; 1.0346x over previous
import jax
import jax.numpy as jnp
from jax.experimental import pallas as pl
from jax.experimental.pallas import tpu as pltpu

INPUT_DIM = 128
TWO_D = 2 * INPUT_DIM
K = 512
B = 512
CHOICE_ALPHA = 0.001
GAMMA = 0.01
NUM_CLASSES = 10
C_PAD = 16

BT = 128
KC = 128
NC = K // KC
G = 2


def _body(x_ref, t_ref, counts_ref, comm_ref, labels_ref,
          out_ref, xn_scr, tT_scr, misc_scr, choice_scr):
    bb = pl.program_id(0)

    @pl.when(bb == 0)
    def _():
        xf = x_ref[...]
        mn = jnp.min(xf)
        mx = jnp.max(xf)
        xn_scr[...] = (xf - mn) / (mx - mn + 1e-10)

        ii = jax.lax.broadcasted_iota(jnp.int32, (K, K), 0)
        jj = jax.lax.broadcasted_iota(jnp.int32, (K, K), 1)
        eye = (ii == jj).astype(jnp.float32)
        tT_scr[...] = jax.lax.dot_general(
            t_ref[...], eye, (((0,), (0,)), ((), ())),
            preferred_element_type=jnp.float32,
        )
        s_t = jnp.sum(t_ref[...], axis=1, keepdims=True)
        cnt = counts_ref[...].astype(jnp.float32)
        cm = comm_ref[...].astype(jnp.float32)
        inv_col = 1.0 / (CHOICE_ALPHA + s_t + GAMMA * cnt)
        stage = jnp.concatenate([inv_col, cm], axis=1)
        misc_scr[0:2, :] = jax.lax.dot_general(
            stage, eye, (((0,), (0,)), ((), ())),
            preferred_element_type=jnp.float32,
        )

    inv_denom = misc_scr[0:1, :]
    comm = misc_scr[1:2, :] > 0.0

    R = BT // 8
    for g in range(R // G):
        accg = [[jnp.zeros((8, KC), dtype=jnp.float32) for _ in range(NC)]
                for _ in range(G)]
        row0 = bb * BT + 8 * G * g
        for d in range(INPUT_DIM):
            a1s = []
            a2s = []
            for j in range(G):
                a1 = jax.lax.broadcast_in_dim(
                    xn_scr[pl.ds(row0 + 8 * j, 8), d : d + 1], (8, KC), (0, 1)
                )
                a1s.append(a1)
                a2s.append(1.0 - a1)
            for c in range(NC):
                b1 = jnp.broadcast_to(tT_scr[d : d + 1, c * KC : (c + 1) * KC], (8, KC))
                b2 = jnp.broadcast_to(
                    tT_scr[d + INPUT_DIM : d + INPUT_DIM + 1, c * KC : (c + 1) * KC],
                    (8, KC),
                )
                for j in range(G):
                    accg[j][c] = (
                        accg[j][c] + jnp.minimum(a1s[j], b1) + jnp.minimum(a2s[j], b2)
                    )
        for j in range(G):
            for c in range(NC):
                cvj = jnp.where(
                    comm[:, c * KC : (c + 1) * KC],
                    accg[j][c] * inv_denom[:, c * KC : (c + 1) * KC],
                    -jnp.inf,
                )
                choice_scr[pl.ds(row0 + 8 * j, 8), c * KC : (c + 1) * KC] = cvj

    @pl.when(bb == pl.num_programs(0) - 1)
    def _():
        cv = choice_scr[...]
        row_max = jnp.max(cv, axis=1, keepdims=True)
        iota_k = jax.lax.broadcasted_iota(jnp.int32, (B, K), 1)
        masked_idx = jnp.where(cv == row_max, iota_k, K)
        best = jnp.min(masked_idx, axis=1, keepdims=True)
        best_oh = (iota_k == best).astype(jnp.float32)

        cls_iota = jax.lax.broadcasted_iota(jnp.int32, (K, C_PAD), 1)
        lab_oh = (labels_ref[...] == cls_iota).astype(jnp.float32)

        cvz = jnp.where(misc_scr[1:2, :] > 0.0, cv, 0.0)
        cls_sums = jnp.dot(cvz, lab_oh, preferred_element_type=jnp.float32)
        pred_oh = jnp.dot(best_oh, lab_oh, preferred_element_type=jnp.float32)
        out_ref[...] = pred_oh * cls_sums


@jax.jit
def _run(x, templates, comm2d, labels2d, counts2d):
    logits_p = pl.pallas_call(
        _body,
        grid=(B // BT,),
        in_specs=[
            pl.BlockSpec((B, INPUT_DIM), lambda bb: (0, 0)),
            pl.BlockSpec((K, TWO_D), lambda bb: (0, 0)),
            pl.BlockSpec((K, 1), lambda bb: (0, 0)),
            pl.BlockSpec((K, 1), lambda bb: (0, 0)),
            pl.BlockSpec((K, 1), lambda bb: (0, 0)),
        ],
        out_specs=pl.BlockSpec((B, C_PAD), lambda bb: (0, 0)),
        out_shape=jax.ShapeDtypeStruct((B, C_PAD), jnp.float32),
        scratch_shapes=[
            pltpu.VMEM((B, INPUT_DIM), jnp.float32),
            pltpu.VMEM((TWO_D, K), jnp.float32),
            pltpu.VMEM((8, K), jnp.float32),
            pltpu.VMEM((B, K), jnp.float32),
        ],
    )(x, templates, counts2d, comm2d, labels2d)
    return logits_p[:, :NUM_CLASSES]


def kernel(x, templates, committed, category_labels, category_counts, num_committed):
    comm2d = committed.reshape(K, 1)
    counts2d = category_counts.reshape(K, 1)
    labels2d = category_labels.reshape(K, 1)
    return _run(x, templates, comm2d, labels2d, counts2d)
